# speedup vs baseline: 1.1308x; 1.1308x over previous
import functools

import jax
import jax.numpy as jnp
from jax import lax
from jax.experimental import pallas as pl
from jax.experimental.pallas import tpu as pltpu
from jax.experimental.pallas import tpu_sc as plsc

_NC = 2
_NS = 16
_NW = _NC * _NS

_D = 4096
_K = 8


def _sc_gather(table, ids):
    n = ids.shape[0]
    b_per_w = n // _NW
    n_chunks = b_per_w // _K

    mesh = plsc.VectorSubcoreMesh(core_axis_name="c", subcore_axis_name="s")

    @functools.partial(
        pl.kernel,
        mesh=mesh,
        out_type=jax.ShapeDtypeStruct((n, _D), jnp.float32),
        scratch_types=[
            pltpu.VMEM((b_per_w,), jnp.int32),
            pltpu.VMEM((_K, _D), jnp.float32),
            pltpu.SemaphoreType.DMA,
        ],
    )
    def k(table_hbm, idx_hbm, out_hbm, idx_v, rows_v, sem):
        wid = lax.axis_index("s") * _NC + lax.axis_index("c")
        base = wid * b_per_w
        pltpu.sync_copy(idx_hbm.at[pl.ds(base, b_per_w)], idx_v)

        def chunk(c, _):
            row0 = c * _K
            pltpu.async_copy(
                table_hbm.at[idx_v.at[pl.ds(row0, _K)]], rows_v, sem
            ).wait()
            pltpu.sync_copy(rows_v, out_hbm.at[pl.ds(base + row0, _K)])
            return 0

        lax.fori_loop(0, n_chunks, chunk, 0)

    return k(table, ids)


def kernel(input_ids, embed_table):
    b, s = input_ids.shape
    ids = input_ids.reshape(-1).astype(jnp.int32)
    rows = _sc_gather(embed_table, ids)
    hidden_state = rows.reshape(b, s, _D)
    attention_mask = jnp.ones((b, s), dtype=jnp.float32)
    return hidden_state, attention_mask


# speedup vs baseline: 1.1665x; 1.0316x over previous
import functools

import jax
import jax.numpy as jnp
from jax import lax
from jax.experimental import pallas as pl
from jax.experimental.pallas import tpu as pltpu
from jax.experimental.pallas import tpu_sc as plsc

_NC = 2
_NS = 16
_NW = _NC * _NS

_D = 4096
_K = 8
_NB = 2


def _sc_gather(table, ids):
    n = ids.shape[0]
    b_per_w = n // _NW
    n_chunks = b_per_w // _K
    n_groups = n_chunks // _NB
    assert n_chunks * _K == b_per_w and n_groups * _NB == n_chunks

    ids2 = ids.reshape(n // _K, _K)

    mesh = plsc.VectorSubcoreMesh(core_axis_name="c", subcore_axis_name="s")

    @functools.partial(
        pl.kernel,
        mesh=mesh,
        out_type=jax.ShapeDtypeStruct((n // _K, _K, _D), jnp.float32),
        scratch_types=[
            pltpu.VMEM((n_chunks, _K), jnp.int32),
            pltpu.VMEM((_NB, _K, _D), jnp.float32),
            [pltpu.SemaphoreType.DMA] * _NB,
            [pltpu.SemaphoreType.DMA] * _NB,
        ],
    )
    def k(table_hbm, idx_hbm, out_hbm, idx_v, rows_v, gsems, ssems):
        wid = lax.axis_index("s") * _NC + lax.axis_index("c")
        chunk0 = wid * n_chunks

        pltpu.sync_copy(idx_hbm.at[pl.ds(chunk0, n_chunks)], idx_v)

        def g_start(c, b):
            pltpu.async_copy(
                table_hbm.at[idx_v.at[c]],
                rows_v.at[b],
                gsems[b],
            )

        def s_start(c, b):
            pltpu.async_copy(rows_v.at[b], out_hbm.at[chunk0 + c], ssems[b])

        def g_wait(b):
            pltpu.make_async_copy(
                table_hbm.at[pl.ds(0, _K)], rows_v.at[b], gsems[b]
            ).wait()

        def s_wait(c, b):
            pltpu.make_async_copy(
                rows_v.at[b], out_hbm.at[chunk0 + c], ssems[b]
            ).wait()

        for b in range(_NB):
            g_start(b, b)

        def group(g, _):
            c0 = g * _NB
            for b in range(_NB):
                g_wait(b)
                s_start(c0 + b, b)
            for b in range(_NB):
                s_wait(c0 + b, b)

                @pl.when(g + 1 < n_groups)
                def _():
                    g_start(c0 + _NB + b, b)

            return 0

        lax.fori_loop(0, n_groups, group, 0)

    return k(table, ids2)


def kernel(input_ids, embed_table):
    b, s = input_ids.shape
    ids = input_ids.reshape(-1).astype(jnp.int32)
    rows = _sc_gather(embed_table, ids)
    hidden_state = rows.reshape(b, s, _D)
    attention_mask = jnp.ones((b, s), dtype=jnp.float32)
    return hidden_state, attention_mask


# speedup vs baseline: 3.2439x; 2.7809x over previous
import functools

import jax
import jax.numpy as jnp
from jax import lax
from jax.experimental import pallas as pl
from jax.experimental.pallas import tpu as pltpu
from jax.experimental.pallas import tpu_sc as plsc

_NC = 2
_NS = 16
_NW = _NC * _NS

_D = 4096
_K = 8
_NB = 2


def _sc_gather(table, ids):
    n = ids.shape[0]
    b_per_w = n // _NW
    n_chunks = b_per_w // _K
    n_groups = n_chunks // _NB
    assert n_chunks * _K == b_per_w and n_groups * _NB == n_chunks

    ids2 = ids.reshape(n // _K, _K)

    mesh = plsc.VectorSubcoreMesh(core_axis_name="c", subcore_axis_name="s")

    @functools.partial(
        pl.kernel,
        mesh=mesh,
        out_type=jax.ShapeDtypeStruct((n // _K, _K, _D), jnp.float32),
        scratch_types=[
            pltpu.VMEM((n_chunks, _K), jnp.int32),
            pltpu.VMEM((_NB, _K, _D), jnp.float32),
            [pltpu.SemaphoreType.DMA] * _NB,
            [pltpu.SemaphoreType.DMA] * _NB,
        ],
    )
    def k(table_hbm, idx_hbm, out_hbm, idx_v, rows_v, gsems, ssems):
        wid = lax.axis_index("s") * _NC + lax.axis_index("c")
        chunk0 = wid * n_chunks

        pltpu.sync_copy(idx_hbm.at[pl.ds(chunk0, n_chunks)], idx_v)

        def g_start(c, b):
            pltpu.async_copy(
                table_hbm.at[idx_v.at[c]],
                rows_v.at[b],
                gsems[b],
            )

        def s_start(c, b):
            pltpu.async_copy(rows_v.at[b], out_hbm.at[chunk0 + c], ssems[b])

        def g_wait(b):
            pltpu.make_async_copy(
                table_hbm.at[pl.ds(0, _K)], rows_v.at[b], gsems[b]
            ).wait()

        def s_wait(c, b):
            pltpu.make_async_copy(
                rows_v.at[b], out_hbm.at[chunk0 + c], ssems[b]
            ).wait()

        for b in range(_NB):
            g_start(b, b)

        def group(g, _):
            c0 = g * _NB
            for b in range(_NB):
                g_wait(b)
                s_start(c0 + b, b)
            for b in range(_NB):
                s_wait(c0 + b, b)

                @pl.when(g + 1 < n_groups)
                def _():
                    g_start(c0 + _NB + b, b)

            return 0

        lax.fori_loop(0, n_groups, group, 0)

    return k(table, ids2)


def kernel(input_ids, embed_table):
    b, s = input_ids.shape
    ids_sm = input_ids.astype(jnp.int32).T.reshape(-1)
    rows = _sc_gather(embed_table, ids_sm)
    hidden_state = rows.reshape(s, b, _D).transpose(1, 0, 2)
    attention_mask = jnp.ones((b, s), dtype=jnp.float32)
    return hidden_state, attention_mask


# speedup vs baseline: 3.2684x; 1.0076x over previous
import functools

import jax
import jax.numpy as jnp
from jax import lax
from jax.experimental import pallas as pl
from jax.experimental.pallas import tpu as pltpu
from jax.experimental.pallas import tpu_sc as plsc

_NC = 2
_NS = 16
_NW = _NC * _NS

_D = 4096
_K = 8
_NB = 3


def _sc_gather(table, ids):
    n = ids.shape[0]
    b_per_w = n // _NW
    n_chunks = b_per_w // _K
    n_groups = n_chunks // _NB
    n_rem = n_chunks - n_groups * _NB
    assert n_chunks * _K == b_per_w

    ids2 = ids.reshape(n // _K, _K)

    mesh = plsc.VectorSubcoreMesh(core_axis_name="c", subcore_axis_name="s")

    @functools.partial(
        pl.kernel,
        mesh=mesh,
        out_type=jax.ShapeDtypeStruct((n // _K, _K, _D), jnp.float32),
        scratch_types=[
            pltpu.VMEM((n_chunks, _K), jnp.int32),
            pltpu.VMEM((_NB, _K, _D), jnp.float32),
            [pltpu.SemaphoreType.DMA] * _NB,
            [pltpu.SemaphoreType.DMA] * _NB,
        ],
    )
    def k(table_hbm, idx_hbm, out_hbm, idx_v, rows_v, gsems, ssems):
        wid = lax.axis_index("s") * _NC + lax.axis_index("c")
        chunk0 = wid * n_chunks

        pltpu.sync_copy(idx_hbm.at[pl.ds(chunk0, n_chunks)], idx_v)

        def g_start(c, b):
            pltpu.async_copy(
                table_hbm.at[idx_v.at[c]],
                rows_v.at[b],
                gsems[b],
            )

        def s_start(c, b):
            pltpu.async_copy(rows_v.at[b], out_hbm.at[chunk0 + c], ssems[b])

        def g_wait(b):
            pltpu.make_async_copy(
                table_hbm.at[pl.ds(0, _K)], rows_v.at[b], gsems[b]
            ).wait()

        def s_wait(c, b):
            pltpu.make_async_copy(
                rows_v.at[b], out_hbm.at[chunk0 + c], ssems[b]
            ).wait()

        for b in range(_NB):
            g_start(b, b)

        def group(g, _):
            c0 = g * _NB
            for b in range(_NB):
                g_wait(b)
                s_start(c0 + b, b)
            for b in range(_NB):
                s_wait(c0 + b, b)

                @pl.when(c0 + _NB + b < n_chunks)
                def _():
                    g_start(c0 + _NB + b, b)

            return 0

        lax.fori_loop(0, n_groups, group, 0)

        c0 = n_groups * _NB
        for b in range(n_rem):
            g_wait(b)
            s_start(c0 + b, b)
        for b in range(n_rem):
            s_wait(c0 + b, b)

    return k(table, ids2)


def kernel(input_ids, embed_table):
    b, s = input_ids.shape
    ids_sm = input_ids.astype(jnp.int32).T.reshape(-1)
    rows = _sc_gather(embed_table, ids_sm)
    hidden_state = rows.reshape(s, b, _D).transpose(1, 0, 2)
    attention_mask = jnp.ones((b, s), dtype=jnp.float32)
    return hidden_state, attention_mask


# speedup vs baseline: 4.4040x; 1.3475x over previous
import functools

import jax
import jax.numpy as jnp
from jax import lax
from jax.experimental import pallas as pl
from jax.experimental.pallas import tpu as pltpu
from jax.experimental.pallas import tpu_sc as plsc

_NC = 2
_NS = 16
_NW = _NC * _NS

_D = 4096
_K = 8
_NB = 2


def _sc_gather(table, ids):
    n = ids.shape[0]
    b_per_w = n // _NW
    n_chunks = b_per_w // _K
    n_pairs = n_chunks // _NB
    assert n_chunks * _K == b_per_w and n_pairs * _NB == n_chunks

    ids16 = ids.reshape(n // 16, 16)

    mesh = plsc.VectorSubcoreMesh(core_axis_name="c", subcore_axis_name="s")

    @functools.partial(
        pl.kernel,
        mesh=mesh,
        out_type=jax.ShapeDtypeStruct((n // _K, _K, _D), jnp.float32),
        scratch_types=[
            pltpu.VMEM((n_pairs, 16), jnp.int32),
            pltpu.VMEM((_NB, _K, _D), jnp.float32),
            pltpu.VMEM_SHARED(table.shape, jnp.float32),
            [pltpu.SemaphoreType.DMA] * _NB,
            [pltpu.SemaphoreType.DMA] * _NB,
        ],
    )
    def k(table_hbm, idx_hbm, out_hbm, idx_v, rows_v, tab_sp, gsems, ssems):
        sid = lax.axis_index("s")
        wid = sid * _NC + lax.axis_index("c")
        chunk0 = wid * n_chunks

        @pl.when(sid == 0)
        def _():
            pltpu.sync_copy(table_hbm, tab_sp)

        plsc.subcore_barrier()

        pltpu.sync_copy(idx_hbm.at[pl.ds(wid * n_pairs, n_pairs)], idx_v)

        def g_start(vec, h):
            for r in range(_K):
                pltpu.async_copy(
                    tab_sp.at[vec[h * _K + r]], rows_v.at[h, r], gsems[h]
                )

        def s_start(c, h):
            pltpu.async_copy(rows_v.at[h], out_hbm.at[chunk0 + c], ssems[h])

        def g_wait(h):
            pltpu.make_async_copy(
                table_hbm.at[pl.ds(0, _K)], rows_v.at[h], gsems[h]
            ).wait()

        def s_wait(c, h):
            pltpu.make_async_copy(
                rows_v.at[h], out_hbm.at[chunk0 + c], ssems[h]
            ).wait()

        vec0 = idx_v[0]
        for h in range(_NB):
            g_start(vec0, h)

        def pair(p, _):
            c0 = p * _NB
            for h in range(_NB):
                g_wait(h)
                s_start(c0 + h, h)
            vec_n = idx_v[lax.min(p + 1, n_pairs - 1)]
            for h in range(_NB):
                s_wait(c0 + h, h)

                @pl.when(p + 1 < n_pairs)
                def _():
                    g_start(vec_n, h)

            return 0

        lax.fori_loop(0, n_pairs, pair, 0)

    return k(table, ids16)


def kernel(input_ids, embed_table):
    b, s = input_ids.shape
    ids_sm = input_ids.astype(jnp.int32).T.reshape(-1)
    rows = _sc_gather(embed_table, ids_sm)
    hidden_state = rows.reshape(s, b, _D).transpose(1, 0, 2)
    attention_mask = jnp.ones((b, s), dtype=jnp.float32)
    return hidden_state, attention_mask
